# baseline (speedup 1.0000x reference)
import functools

import jax
import jax.numpy as jnp
from jax import lax
from jax.experimental import pallas as pl
from jax.experimental.pallas import tpu as pltpu
from jax.experimental.pallas import tpu_sc as plsc

_WINDOW = 12


def kernel(scan, row_weight, col_weight, depth_weight):
    d, em, h, w = scan.shape
    info = plsc.get_sparse_core_info()
    nc, ns = info.num_cores, info.num_subcores
    nw = nc * ns
    per_w = (d * h) // nw
    h_per_w = h // nw
    dq = nw // d
    h_per_dq = h // dq
    mesh = plsc.VectorSubcoreMesh(core_axis_name="c", subcore_axis_name="s")

    @functools.partial(
        pl.kernel, mesh=mesh,
        compiler_params=pltpu.CompilerParams(use_tc_tiling_on_sc=False),
        out_type=jax.ShapeDtypeStruct((d, h, w, 192), jnp.float32),
        scratch_types=[
            pltpu.VMEM((w, 64), jnp.float32),
            pltpu.VMEM((h, 64), jnp.float32),
            pltpu.VMEM((d, 64), jnp.float32),
            pltpu.VMEM((w, 64), jnp.float32),
            pltpu.VMEM((w, 64), jnp.float32),
            pltpu.SemaphoreType.DMA((3,)),
        ],
    )
    def sc_k(row_hbm, col_hbm, depth_hbm, out_hbm, col_v, rowtab_v, depthtab_v,
             rowsplat_v, depthsplat_v, sems):
        wid = lax.axis_index("s") * nc + lax.axis_index("c")
        pltpu.sync_copy(col_hbm.at[pl.ds(0, w)], col_v)
        pltpu.sync_copy(row_hbm.at[pl.ds(0, h)], rowtab_v)
        pltpu.sync_copy(depth_hbm.at[pl.ds(0, d)], depthtab_v)

        def col_dst(s):
            di = s // h
            hi = s - di * h
            return out_hbm.at[di, hi, :, pl.ds(0, 64)]

        def col_copy(s):
            return pltpu.make_async_copy(col_v, col_dst(s), sems.at[0])

        base = wid * per_w

        def col_loop(i, carry):
            col_copy(base + i).start()

            @pl.when(i >= _WINDOW)
            def _():
                col_copy(base + i).wait()
            return carry

        lax.fori_loop(0, per_w, col_loop, 0)
        for j in range(_WINDOW):
            col_copy(base + j).wait()

        def row_copy(hi, di):
            return pltpu.make_async_copy(
                rowsplat_v, out_hbm.at[di, hi, :, pl.ds(64, 64)], sems.at[1])

        def row_h(j, carry):
            hi = wid * h_per_w + j
            r = [rowtab_v[hi, pl.ds(k * 16, 16)] for k in range(4)]

            def fill(ww, c2):
                for k in range(4):
                    rowsplat_v[ww, pl.ds(k * 16, 16)] = r[k]
                return c2

            lax.fori_loop(0, w, fill, 0)
            for di in range(d):
                row_copy(hi, di).start()
            for di in range(d):
                row_copy(hi, di).wait()
            return carry

        lax.fori_loop(0, h_per_w, row_h, 0)

        dmine = lax.rem(wid, d)
        q = wid // d
        dpv = [depthtab_v[dmine, pl.ds(k * 16, 16)] for k in range(4)]

        def dfill(ww, c2):
            for k in range(4):
                depthsplat_v[ww, pl.ds(k * 16, 16)] = dpv[k]
            return c2

        lax.fori_loop(0, w, dfill, 0)

        def depth_copy(hi):
            return pltpu.make_async_copy(
                depthsplat_v, out_hbm.at[dmine, hi, :, pl.ds(128, 64)],
                sems.at[2])

        def depth_loop(i, carry):
            depth_copy(q * h_per_dq + i).start()

            @pl.when(i >= _WINDOW)
            def _():
                depth_copy(q * h_per_dq + i).wait()
            return carry

        lax.fori_loop(0, h_per_dq, depth_loop, 0)
        for j in range(_WINDOW):
            depth_copy(q * h_per_dq + j).wait()

    return sc_k(row_weight, col_weight, depth_weight)


# speedup vs baseline: 3.9128x; 3.9128x over previous
import functools

import jax
import jax.numpy as jnp
from jax.experimental import pallas as pl
from jax.experimental.pallas import tpu as pltpu


def _pos_body(row_ref, col_ref, depth_ref, out_ref, *, hb, w):
    di = pl.program_id(0)
    col = col_ref[0:w, :]
    row = row_ref[...]
    depth = depth_ref[pl.ds(di, 1), :]
    zc = jnp.zeros((w, 64), jnp.float32)
    zr = jnp.zeros((hb, 64), jnp.float32)
    u = jnp.concatenate(
        [col, zc, jnp.broadcast_to(depth, (w, 64))], axis=-1)
    v = jnp.concatenate([zr, row, zr], axis=-1)
    out_ref[...] = (u[None, :, :] + v[:, None, :])[None]


def kernel(scan, row_weight, col_weight, depth_weight):
    d, em, h, w = scan.shape
    hb = 32
    n_h = h // hb
    body = functools.partial(_pos_body, hb=hb, w=w)
    out = pl.pallas_call(
        body,
        grid=(d, n_h),
        in_specs=[
            pl.BlockSpec((hb, 64), lambda di, hi: (hi, 0)),
            pl.BlockSpec((256, 64), lambda di, hi: (0, 0)),
            pl.BlockSpec((40, 64), lambda di, hi: (0, 0)),
        ],
        out_specs=pl.BlockSpec((1, hb, w, 192), lambda di, hi: (di, hi, 0, 0)),
        out_shape=jax.ShapeDtypeStruct((d, h, w, 192), jnp.float32),
        compiler_params=pltpu.CompilerParams(
            dimension_semantics=("parallel", "parallel")),
    )(row_weight, col_weight, depth_weight)
    return out
